# Initial kernel scaffold; baseline (speedup 1.0000x reference)
import jax
import jax.numpy as jnp
from jax.experimental import pallas as pl


def kernel(x, c1_W, c1_b, bn1_g, bn1_b, c2_W, c2_b, bn2_g, bn2_b, c3_W, c3_b, bn3_g, bn3_b, c4_W, c4_b, bn4_g, bn4_b, c5_W, c5_b, bn5_g, bn5_b, sc1_W, sc1_b, sbn1_g, sbn1_b, sc2_W, sc2_b, sbn2_g, sbn2_b, sfc1_W, sfc1_b, sfc2_W, sfc2_b, h_Wih1, h_Whh1, h_bih1, h_bhh1, h_Wih2, h_Whh2, h_bih2, h_bhh2, h_fcW, h_fcb):
    raise NotImplementedError("write your pallas kernel here")


# speedup vs baseline: 1.3023x; 1.3023x over previous
import functools

import jax
import jax.numpy as jnp
from jax.experimental import pallas as pl
from jax.experimental.pallas import tpu as pltpu

_LEAK = 0.1
_EPS = 1e-5
_B = 32
_T = 32
_HID = 256
_A1 = 129
_NS = 4


def _dg2(a, b):
    return jax.lax.dot_general(a, b, (((a.ndim - 1,), (1,)), ((), ())),
                               preferred_element_type=jnp.float32)


def _leaky(x):
    return jnp.where(x >= 0, x, _LEAK * x)


def _bn(y, g, bb, axes):
    m = jnp.mean(y, axis=axes, keepdims=True)
    v = jnp.mean((y - m) * (y - m), axis=axes, keepdims=True)
    return g * (y - m) / jnp.sqrt(v + _EPS) + bb


def _conv1_kernel(x_ref, w_ref, b_ref, g_ref, bb_ref, o_ref):
    acc = jnp.zeros((_B, 32, 256), jnp.float32)
    for dh in range(3):
        for dw in range(3):
            k = 3 * dh + dw
            acc = acc + x_ref[:, dh:dh + 32, dw:dw + 256] * w_ref[0][0:1, k:k + 1]
    y = _leaky(acc + b_ref[0, 0, 0])
    m = jnp.mean(y)
    v = jnp.mean((y - m) * (y - m))
    y = g_ref[0, 0, 0] * (y - m) / jnp.sqrt(v + _EPS) + bb_ref[0, 0, 0]
    y = jnp.max(y.reshape(_B, 16, 2, 256), axis=2)
    o_ref[...] = y[None]


def _conv1(x3, c1_W, c1_b, bn1_g, bn1_b):
    return pl.pallas_call(
        _conv1_kernel,
        grid=(16,),
        in_specs=[
            pl.BlockSpec((_B, 34, 258), lambda c: (0, 0, 0)),
            pl.BlockSpec((1, 1, 9), lambda c: (c, 0, 0)),
            pl.BlockSpec((1, 1, 1), lambda c: (c, 0, 0)),
            pl.BlockSpec((1, 1, 1), lambda c: (c, 0, 0)),
            pl.BlockSpec((1, 1, 1), lambda c: (c, 0, 0)),
        ],
        out_specs=pl.BlockSpec((1, _B, 16, 256), lambda c: (c, 0, 0, 0)),
        out_shape=jax.ShapeDtypeStruct((16, _B, 16, 256), jnp.float32),
    )(x3, c1_W.reshape(16, 1, 9), c1_b.reshape(16, 1, 1),
      bn1_g.reshape(16, 1, 1), bn1_b.reshape(16, 1, 1))


def _conv_bhw_kernel(x_ref, w_ref, b_ref, g_ref, bb_ref, o_ref, *, H, W, Cin,
                     Cout, parity_in):
    if parity_in:
        xm = jnp.maximum(x_ref[:, :, 0], x_ref[:, :, 1])
    else:
        xm = x_ref[...]
    acc = jnp.zeros((_B * H, Cout, W), jnp.float32)
    for dh in range(3):
        for dw in range(3):
            xs = xm[:, dh:dh + H, :, dw:dw + W].reshape(_B * H, Cin, W)
            wb = jnp.broadcast_to(w_ref[dh, dw][None], (_B * H, Cout, Cin))
            acc = acc + jax.lax.dot_general(
                wb, xs, (((2,), (1,)), ((0,), (0,))),
                preferred_element_type=jnp.float32)
    y = _bn(_leaky(acc + b_ref[...]), g_ref[...], bb_ref[...], (0, 2))
    y = y.reshape(_B, H, Cout, W).reshape(_B, H // 2, 2, Cout, W)
    o_ref[...] = jnp.max(y, axis=2)


def _conv_bhw(x_pad, W_oihw, b, g, bb, H, W, parity_in):
    Cout, Cin = W_oihw.shape[0], W_oihw.shape[1]
    w = W_oihw.transpose(2, 3, 0, 1)
    return pl.pallas_call(
        functools.partial(_conv_bhw_kernel, H=H, W=W, Cin=Cin, Cout=Cout,
                          parity_in=parity_in),
        out_shape=jax.ShapeDtypeStruct((_B, H // 2, Cout, W), jnp.float32),
    )(x_pad, w, b.reshape(1, -1, 1), g.reshape(1, -1, 1), bb.reshape(1, -1, 1))


def _conv_nhwc_kernel(x_ref, w_ref, b_ref, g_ref, bb_ref, o_ref, pad_sc, *,
                      H, W, Cin, Cout):
    xin = jnp.maximum(x_ref[..., :Cin], x_ref[..., Cin:])
    pad_sc[...] = jnp.zeros((_B, H + 2, W + 2, Cin), jnp.float32)
    pad_sc[:, 1:H + 1, 1:W + 1, :] = xin
    x = pad_sc[...]
    acc = jnp.zeros((_B, H, W, Cout), jnp.float32)
    for dh in range(3):
        for dw in range(3):
            xs = x[:, dh:dh + H, dw:dw + W, :]
            acc = acc + jax.lax.dot_general(
                xs, w_ref[dh, dw], (((3,), (0,)), ((), ())),
                preferred_element_type=jnp.float32)
    y = _bn(_leaky(acc + b_ref[...]), g_ref[...], bb_ref[...], (0, 1, 2))
    y = jnp.max(y.reshape(_B, H // 2, 2, W, Cout), axis=2)
    o_ref[...] = y


def _conv_nhwc(x_packed, W_oihw, b, g, bb, H, W):
    Cout, Cin = W_oihw.shape[0], W_oihw.shape[1]
    w = W_oihw.transpose(2, 3, 1, 0)
    return pl.pallas_call(
        functools.partial(_conv_nhwc_kernel, H=H, W=W, Cin=Cin, Cout=Cout),
        out_shape=jax.ShapeDtypeStruct((_B, H // 2, W, Cout), jnp.float32),
        scratch_shapes=[pltpu.VMEM((_B, H + 2, W + 2, Cin), jnp.float32)],
    )(x_packed, w, b.reshape(1, 1, 1, -1), g.reshape(1, 1, 1, -1),
      bb.reshape(1, 1, 1, -1))


def _classifier_kernel(x_ref, w1_ref, b1_ref, g1_ref, bb1_ref, w2_ref, b2_ref,
                       g2_ref, bb2_ref, f1w_ref, f1b_ref, f2w_ref, f2b_ref,
                       o_ref):
    x = x_ref[...]
    acc = jnp.zeros((_B, 29, 128), jnp.float32)
    for j in range(4):
        acc = acc + jax.lax.dot_general(
            x[:, j:j + 29, :], w1_ref[j], (((2,), (0,)), ((), ())),
            preferred_element_type=jnp.float32)
    y = _bn(_leaky(acc + b1_ref[...]), g1_ref[...], bb1_ref[...], (0, 1))
    y = jnp.concatenate(
        [jnp.max(y[:, 4 * k:4 * k + 4, :], axis=1, keepdims=True)
         for k in range(7)], axis=1)
    acc2 = jnp.zeros((_B, 4, 64), jnp.float32)
    for j in range(4):
        acc2 = acc2 + jax.lax.dot_general(
            y[:, j:j + 4, :], w2_ref[j], (((2,), (0,)), ((), ())),
            preferred_element_type=jnp.float32)
    z = _bn(_leaky(acc2 + b2_ref[...]), g2_ref[...], bb2_ref[...], (0, 1))
    z = jnp.max(z, axis=1)
    h = jnp.maximum(_dg2(z, f1w_ref[...]) + f1b_ref[...], 0.0)
    logits = _dg2(h, f2w_ref[...]) + f2b_ref[...]
    mx = jnp.max(logits, axis=1, keepdims=True)
    e = jnp.exp(logits - mx)
    o_ref[...] = e / jnp.sum(e, axis=1, keepdims=True)


def _classifier(xp, sc1_W, sc1_b, sbn1_g, sbn1_b, sc2_W, sc2_b, sbn2_g,
                sbn2_b, sfc1_W, sfc1_b, sfc2_W, sfc2_b):
    return pl.pallas_call(
        _classifier_kernel,
        out_shape=jax.ShapeDtypeStruct((_B, _NS), jnp.float32),
    )(xp, sc1_W.transpose(2, 1, 0), sc1_b.reshape(1, 1, -1),
      sbn1_g.reshape(1, 1, -1), sbn1_b.reshape(1, 1, -1),
      sc2_W.transpose(2, 1, 0), sc2_b.reshape(1, 1, -1),
      sbn2_g.reshape(1, 1, -1), sbn2_b.reshape(1, 1, -1),
      sfc1_W, sfc1_b.reshape(1, -1), sfc2_W, sfc2_b.reshape(1, -1))


def _lstm_layer(ih_sc, y_sc, whh, h0, c0):
    def step(t, hc):
        h, c = hc
        g = ih_sc[pl.ds(t * _B, _B), :] + _dg2(h, whh)
        i = jax.nn.sigmoid(g[:, 0:256])
        f = jax.nn.sigmoid(g[:, 256:512])
        gg = jnp.tanh(g[:, 512:768])
        o = jax.nn.sigmoid(g[:, 768:1024])
        c = f * c + i * gg
        h = o * jnp.tanh(c)
        y_sc[pl.ds(t * _B, _B), :] = h
        return (h, c)

    return jax.lax.fori_loop(0, _T, step, (h0, c0))


def _heads_kernel(xs_ref, p_ref, wih1_ref, whh1_ref, bih1_ref, bhh1_ref,
                  wih2_ref, whh2_ref, bih2_ref, bhh2_ref, fcw_ref, fcb_ref,
                  o_ref, ih_sc, y_sc):
    s = pl.program_id(0)
    X = xs_ref[...]
    ih_sc[...] = _dg2(X, wih1_ref[0]) + (bih1_ref[0] + bhh1_ref[0])
    z = jnp.zeros((_B, _HID), jnp.float32)
    h1, c1 = _lstm_layer(ih_sc, y_sc, whh1_ref[0], z, z)

    ih_sc[...] = _dg2(y_sc[...], wih2_ref[0]) + (bih2_ref[0] + bhh2_ref[0])
    _lstm_layer(ih_sc, y_sc, whh2_ref[0], h1, c1)

    logits = _dg2(y_sc[...], fcw_ref[0]) + fcb_ref[0]
    mx = jnp.max(logits, axis=1, keepdims=True)
    e = jnp.exp(logits - mx)
    probs = e / jnp.sum(e, axis=1, keepdims=True)

    p = p_ref[...]
    pm = jnp.max(p, axis=1, keepdims=True)
    iota = jax.lax.broadcasted_iota(jnp.int32, (_B, _NS), 1)
    sidx = jnp.min(jnp.where(p >= pm, iota, _NS), axis=1, keepdims=True)
    maskb = (sidx == s)

    o3 = probs.reshape(_T, _B, _A1)
    o_ref[...] = jnp.where(maskb[None], o3, o_ref[...])


def _heads(xs2, probs, h_Wih1, h_Whh1, h_bih1, h_bhh1, h_Wih2, h_Whh2,
           h_bih2, h_bhh2, h_fcW, h_fcb):
    full = lambda shape: pl.BlockSpec(shape, lambda s: (0,) * len(shape))
    per_head = lambda shape: pl.BlockSpec(
        (1,) + shape, lambda s, _shape=shape: (s,) + (0,) * len(_shape))
    return pl.pallas_call(
        _heads_kernel,
        grid=(_NS,),
        in_specs=[
            full((_T * _B, _HID)),
            full((_B, _NS)),
            per_head((4 * _HID, _HID)),
            per_head((4 * _HID, _HID)),
            per_head((1, 4 * _HID)),
            per_head((1, 4 * _HID)),
            per_head((4 * _HID, _HID)),
            per_head((4 * _HID, _HID)),
            per_head((1, 4 * _HID)),
            per_head((1, 4 * _HID)),
            per_head((_A1, _HID)),
            per_head((1, _A1)),
        ],
        out_specs=pl.BlockSpec((_T, _B, _A1), lambda s: (0, 0, 0)),
        out_shape=jax.ShapeDtypeStruct((_T, _B, _A1), jnp.float32),
        scratch_shapes=[
            pltpu.VMEM((_T * _B, 4 * _HID), jnp.float32),
            pltpu.VMEM((_T * _B, _HID), jnp.float32),
        ],
    )(xs2, probs, h_Wih1, h_Whh1, h_bih1.reshape(_NS, 1, -1),
      h_bhh1.reshape(_NS, 1, -1), h_Wih2, h_Whh2, h_bih2.reshape(_NS, 1, -1),
      h_bhh2.reshape(_NS, 1, -1), h_fcW, h_fcb.reshape(_NS, 1, -1))


def kernel(x, c1_W, c1_b, bn1_g, bn1_b, c2_W, c2_b, bn2_g, bn2_b, c3_W, c3_b,
           bn3_g, bn3_b, c4_W, c4_b, bn4_g, bn4_b, c5_W, c5_b, bn5_g, bn5_b,
           sc1_W, sc1_b, sbn1_g, sbn1_b, sc2_W, sc2_b, sbn2_g, sbn2_b, sfc1_W,
           sfc1_b, sfc2_W, sfc2_b, h_Wih1, h_Whh1, h_bih1, h_bhh1, h_Wih2,
           h_Whh2, h_bih2, h_bhh2, h_fcW, h_fcb):
    x3 = jnp.pad(x[:, 0], ((0, 0), (1, 1), (1, 1)))
    y1 = _conv1(x3, c1_W, c1_b, bn1_g, bn1_b)
    x2 = y1.transpose(1, 2, 0, 3).reshape(_B, 16, 16, 128, 2)
    x2 = x2.transpose(0, 1, 4, 2, 3)
    x2 = jnp.pad(x2, ((0, 0), (1, 1), (0, 0), (0, 0), (1, 1)))
    y2 = _conv_bhw(x2, c2_W, c2_b, bn2_g, bn2_b, 16, 128, True)
    x3b = jnp.pad(y2, ((0, 0), (1, 1), (0, 0), (1, 1)))
    y3 = _conv_bhw(x3b, c3_W, c3_b, bn3_g, bn3_b, 8, 128, False)
    x4 = y3.transpose(0, 1, 3, 2).reshape(_B, 4, 64, 2, 64)
    x4 = x4.reshape(_B, 4, 64, 128)
    y4 = _conv_nhwc(x4, c4_W, c4_b, bn4_g, bn4_b, 4, 64)
    x5 = y4.reshape(_B, 2, 32, 2, 128).reshape(_B, 2, 32, 256)
    y5 = _conv_nhwc(x5, c5_W, c5_b, bn5_g, bn5_b, 2, 32)
    xp = y5.reshape(_B, _T, 256)

    probs = _classifier(xp, sc1_W, sc1_b, sbn1_g, sbn1_b, sc2_W, sc2_b,
                        sbn2_g, sbn2_b, sfc1_W, sfc1_b, sfc2_W, sfc2_b)

    xs2 = xp.transpose(1, 0, 2).reshape(_T * _B, 256)
    out_tba = _heads(xs2, probs, h_Wih1, h_Whh1, h_bih1, h_bhh1, h_Wih2,
                     h_Whh2, h_bih2, h_bhh2, h_fcW, h_fcb)
    output = out_tba.transpose(1, 0, 2)
    return output, probs


# speedup vs baseline: 1.3997x; 1.0748x over previous
import functools

import jax
import jax.numpy as jnp
from jax.experimental import pallas as pl
from jax.experimental.pallas import tpu as pltpu

_LEAK = 0.1
_EPS = 1e-5
_B = 32
_T = 32
_HID = 256
_A1 = 129
_NS = 4


def _dg2(a, b):
    return jax.lax.dot_general(a, b, (((a.ndim - 1,), (1,)), ((), ())),
                               preferred_element_type=jnp.float32)


def _leaky(x):
    return jnp.where(x >= 0, x, _LEAK * x)


def _bn(y, g, bb, axes):
    m = jnp.mean(y, axis=axes, keepdims=True)
    v = jnp.mean((y - m) * (y - m), axis=axes, keepdims=True)
    return g * (y - m) / jnp.sqrt(v + _EPS) + bb


def _conv1_kernel(x_ref, w_ref, b_ref, g_ref, bb_ref, o_ref):
    acc = jnp.zeros((_B, 32, 256), jnp.float32)
    for dh in range(3):
        for dw in range(3):
            k = 3 * dh + dw
            acc = acc + x_ref[:, dh:dh + 32, dw:dw + 256] * w_ref[0][0:1, k:k + 1]
    y = _leaky(acc + b_ref[0, 0, 0])
    m = jnp.mean(y)
    v = jnp.mean((y - m) * (y - m))
    y = g_ref[0, 0, 0] * (y - m) / jnp.sqrt(v + _EPS) + bb_ref[0, 0, 0]
    y = jnp.max(y.reshape(_B, 16, 2, 256), axis=2)
    o_ref[...] = y[None]


def _conv1(x3, c1_W, c1_b, bn1_g, bn1_b):
    return pl.pallas_call(
        _conv1_kernel,
        grid=(16,),
        in_specs=[
            pl.BlockSpec((_B, 34, 258), lambda c: (0, 0, 0)),
            pl.BlockSpec((1, 1, 9), lambda c: (c, 0, 0)),
            pl.BlockSpec((1, 1, 1), lambda c: (c, 0, 0)),
            pl.BlockSpec((1, 1, 1), lambda c: (c, 0, 0)),
            pl.BlockSpec((1, 1, 1), lambda c: (c, 0, 0)),
        ],
        out_specs=pl.BlockSpec((1, _B, 16, 256), lambda c: (c, 0, 0, 0)),
        out_shape=jax.ShapeDtypeStruct((16, _B, 16, 256), jnp.float32),
    )(x3, c1_W.reshape(16, 1, 9), c1_b.reshape(16, 1, 1),
      bn1_g.reshape(16, 1, 1), bn1_b.reshape(16, 1, 1))


def _conv_bhw_kernel(x_ref, w_ref, b_ref, g_ref, bb_ref, o_ref, *, H, W, Cin,
                     Cout, parity_in):
    if parity_in:
        xm = jnp.maximum(x_ref[:, :, 0], x_ref[:, :, 1])
    else:
        xm = x_ref[...]
    acc = jnp.zeros((_B * H, Cout, W), jnp.float32)
    for dh in range(3):
        for dw in range(3):
            xs = xm[:, dh:dh + H, :, dw:dw + W].reshape(_B * H, Cin, W)
            wb = jnp.broadcast_to(w_ref[dh, dw][None], (_B * H, Cout, Cin))
            acc = acc + jax.lax.dot_general(
                wb, xs, (((2,), (1,)), ((0,), (0,))),
                preferred_element_type=jnp.float32)
    y = _bn(_leaky(acc + b_ref[...]), g_ref[...], bb_ref[...], (0, 2))
    y = y.reshape(_B, H, Cout, W).reshape(_B, H // 2, 2, Cout, W)
    o_ref[...] = jnp.max(y, axis=2)


def _conv_bhw(x_pad, W_oihw, b, g, bb, H, W, parity_in):
    Cout, Cin = W_oihw.shape[0], W_oihw.shape[1]
    w = W_oihw.transpose(2, 3, 0, 1)
    return pl.pallas_call(
        functools.partial(_conv_bhw_kernel, H=H, W=W, Cin=Cin, Cout=Cout,
                          parity_in=parity_in),
        out_shape=jax.ShapeDtypeStruct((_B, H // 2, Cout, W), jnp.float32),
    )(x_pad, w, b.reshape(1, -1, 1), g.reshape(1, -1, 1), bb.reshape(1, -1, 1))


def _conv_nhwc_kernel(x_ref, w_ref, b_ref, g_ref, bb_ref, o_ref, pad_sc, *,
                      H, W, Cin, Cout):
    xin = jnp.maximum(x_ref[..., :Cin], x_ref[..., Cin:])
    pad_sc[...] = jnp.zeros((_B, H + 2, W + 2, Cin), jnp.float32)
    pad_sc[:, 1:H + 1, 1:W + 1, :] = xin
    x = pad_sc[...]
    acc = jnp.zeros((_B, H, W, Cout), jnp.float32)
    for dh in range(3):
        for dw in range(3):
            xs = x[:, dh:dh + H, dw:dw + W, :]
            acc = acc + jax.lax.dot_general(
                xs, w_ref[dh, dw], (((3,), (0,)), ((), ())),
                preferred_element_type=jnp.float32)
    y = _bn(_leaky(acc + b_ref[...]), g_ref[...], bb_ref[...], (0, 1, 2))
    y = jnp.max(y.reshape(_B, H // 2, 2, W, Cout), axis=2)
    o_ref[...] = y


def _conv_nhwc(x_packed, W_oihw, b, g, bb, H, W):
    Cout, Cin = W_oihw.shape[0], W_oihw.shape[1]
    w = W_oihw.transpose(2, 3, 1, 0)
    return pl.pallas_call(
        functools.partial(_conv_nhwc_kernel, H=H, W=W, Cin=Cin, Cout=Cout),
        out_shape=jax.ShapeDtypeStruct((_B, H // 2, W, Cout), jnp.float32),
        scratch_shapes=[pltpu.VMEM((_B, H + 2, W + 2, Cin), jnp.float32)],
    )(x_packed, w, b.reshape(1, 1, 1, -1), g.reshape(1, 1, 1, -1),
      bb.reshape(1, 1, 1, -1))


def _classifier_kernel(x_ref, w1_ref, b1_ref, g1_ref, bb1_ref, w2_ref, b2_ref,
                       g2_ref, bb2_ref, f1w_ref, f1b_ref, f2w_ref, f2b_ref,
                       o_ref):
    x = x_ref[...]
    acc = jnp.zeros((_B, 29, 128), jnp.float32)
    for j in range(4):
        acc = acc + jax.lax.dot_general(
            x[:, j:j + 29, :], w1_ref[j], (((2,), (0,)), ((), ())),
            preferred_element_type=jnp.float32)
    y = _bn(_leaky(acc + b1_ref[...]), g1_ref[...], bb1_ref[...], (0, 1))
    y = jnp.concatenate(
        [jnp.max(y[:, 4 * k:4 * k + 4, :], axis=1, keepdims=True)
         for k in range(7)], axis=1)
    acc2 = jnp.zeros((_B, 4, 64), jnp.float32)
    for j in range(4):
        acc2 = acc2 + jax.lax.dot_general(
            y[:, j:j + 4, :], w2_ref[j], (((2,), (0,)), ((), ())),
            preferred_element_type=jnp.float32)
    z = _bn(_leaky(acc2 + b2_ref[...]), g2_ref[...], bb2_ref[...], (0, 1))
    z = jnp.max(z, axis=1)
    h = jnp.maximum(_dg2(z, f1w_ref[...]) + f1b_ref[...], 0.0)
    logits = _dg2(h, f2w_ref[...]) + f2b_ref[...]
    mx = jnp.max(logits, axis=1, keepdims=True)
    e = jnp.exp(logits - mx)
    o_ref[...] = e / jnp.sum(e, axis=1, keepdims=True)


def _classifier(xp, sc1_W, sc1_b, sbn1_g, sbn1_b, sc2_W, sc2_b, sbn2_g,
                sbn2_b, sfc1_W, sfc1_b, sfc2_W, sfc2_b):
    return pl.pallas_call(
        _classifier_kernel,
        out_shape=jax.ShapeDtypeStruct((_B, _NS), jnp.float32),
    )(xp, sc1_W.transpose(2, 1, 0), sc1_b.reshape(1, 1, -1),
      sbn1_g.reshape(1, 1, -1), sbn1_b.reshape(1, 1, -1),
      sc2_W.transpose(2, 1, 0), sc2_b.reshape(1, 1, -1),
      sbn2_g.reshape(1, 1, -1), sbn2_b.reshape(1, 1, -1),
      sfc1_W, sfc1_b.reshape(1, -1), sfc2_W, sfc2_b.reshape(1, -1))


def _heads_kernel(xs_ref, p_ref, wih1_ref, whh1_ref, b1_ref, wih2_ref,
                  whh2_ref, b2_ref, fcw_ref, fcb_ref, o_ref, ih_sc, y_sc):
    N = _T * _B
    X = xs_ref[...]

    p = p_ref[...]
    pm = jnp.max(p, axis=1, keepdims=True)
    iota = jax.lax.broadcasted_iota(jnp.int32, (_B, _NS), 1)
    sidx = jnp.min(jnp.where(p >= pm, iota, _NS), axis=1, keepdims=True)
    ri = jax.lax.broadcasted_iota(jnp.int32, (N, _B), 0)
    bj = jax.lax.broadcasted_iota(jnp.int32, (N, _B), 1)
    onehot = ((ri % _B) == bj).astype(jnp.float32)
    sidx_rows = jax.lax.dot_general(
        onehot, sidx.astype(jnp.float32), (((1,), (0,)), ((), ())),
        preferred_element_type=jnp.float32)

    for s in range(4):
        ih_sc[s * N:(s + 1) * N, :] = _dg2(X, wih1_ref[s]) + b1_ref[s:s + 1]

    def step1(t, hc):
        h, c = hc
        g = jnp.concatenate(
            [ih_sc[pl.ds(s * N + t * _B, _B), :] +
             _dg2(h[s * _B:(s + 1) * _B], whh1_ref[s]) for s in range(4)],
            axis=0)
        i = jax.nn.sigmoid(g[:, 0:256])
        f = jax.nn.sigmoid(g[:, 256:512])
        gg = jnp.tanh(g[:, 512:768])
        o = jax.nn.sigmoid(g[:, 768:1024])
        c = f * c + i * gg
        h = o * jnp.tanh(c)
        for s in range(4):
            y_sc[pl.ds(s * N + t * _B, _B), :] = h[s * _B:(s + 1) * _B]
        return (h, c)

    z = jnp.zeros((4 * _B, _HID), jnp.float32)
    h1, c1 = jax.lax.fori_loop(0, _T, step1, (z, z))

    for s in range(4):
        ih_sc[s * N:(s + 1) * N, :] = (
            _dg2(y_sc[s * N:(s + 1) * N, :], wih2_ref[s]) + b2_ref[s:s + 1])

    def step2(t, hc):
        h, c = hc
        g = jnp.concatenate(
            [ih_sc[pl.ds(s * N + t * _B, _B), :] +
             _dg2(h[s * _B:(s + 1) * _B], whh2_ref[s]) for s in range(4)],
            axis=0)
        i = jax.nn.sigmoid(g[:, 0:256])
        f = jax.nn.sigmoid(g[:, 256:512])
        gg = jnp.tanh(g[:, 512:768])
        o = jax.nn.sigmoid(g[:, 768:1024])
        c = f * c + i * gg
        h = o * jnp.tanh(c)
        for s in range(4):
            y_sc[pl.ds(s * N + t * _B, _B), :] = h[s * _B:(s + 1) * _B]
        return (h, c)

    jax.lax.fori_loop(0, _T, step2, (h1, c1))

    logits = jnp.zeros((N, _A1), jnp.float32)
    for s in range(4):
        ls = _dg2(y_sc[s * N:(s + 1) * N, :], fcw_ref[s]) + fcb_ref[s:s + 1]
        logits = jnp.where(sidx_rows == float(s), ls, logits)
    mx = jnp.max(logits, axis=1, keepdims=True)
    e = jnp.exp(logits - mx)
    o_ref[...] = e / jnp.sum(e, axis=1, keepdims=True)


def _heads(xs2, probs, h_Wih1, h_Whh1, h_bih1, h_bhh1, h_Wih2, h_Whh2,
           h_bih2, h_bhh2, h_fcW, h_fcb):
    N = _T * _B
    return pl.pallas_call(
        _heads_kernel,
        out_shape=jax.ShapeDtypeStruct((N, _A1), jnp.float32),
        scratch_shapes=[
            pltpu.VMEM((_NS * N, 4 * _HID), jnp.float32),
            pltpu.VMEM((_NS * N, _HID), jnp.float32),
        ],
    )(xs2, probs, h_Wih1, h_Whh1, h_bih1 + h_bhh1, h_Wih2, h_Whh2,
      h_bih2 + h_bhh2, h_fcW, h_fcb)


def kernel(x, c1_W, c1_b, bn1_g, bn1_b, c2_W, c2_b, bn2_g, bn2_b, c3_W, c3_b,
           bn3_g, bn3_b, c4_W, c4_b, bn4_g, bn4_b, c5_W, c5_b, bn5_g, bn5_b,
           sc1_W, sc1_b, sbn1_g, sbn1_b, sc2_W, sc2_b, sbn2_g, sbn2_b, sfc1_W,
           sfc1_b, sfc2_W, sfc2_b, h_Wih1, h_Whh1, h_bih1, h_bhh1, h_Wih2,
           h_Whh2, h_bih2, h_bhh2, h_fcW, h_fcb):
    x3 = jnp.pad(x[:, 0], ((0, 0), (1, 1), (1, 1)))
    y1 = _conv1(x3, c1_W, c1_b, bn1_g, bn1_b)
    x2 = y1.transpose(1, 2, 0, 3).reshape(_B, 16, 16, 128, 2)
    x2 = x2.transpose(0, 1, 4, 2, 3)
    x2 = jnp.pad(x2, ((0, 0), (1, 1), (0, 0), (0, 0), (1, 1)))
    y2 = _conv_bhw(x2, c2_W, c2_b, bn2_g, bn2_b, 16, 128, True)
    x3b = jnp.pad(y2, ((0, 0), (1, 1), (0, 0), (1, 1)))
    y3 = _conv_bhw(x3b, c3_W, c3_b, bn3_g, bn3_b, 8, 128, False)
    x4 = y3.transpose(0, 1, 3, 2).reshape(_B, 4, 64, 2, 64)
    x4 = x4.reshape(_B, 4, 64, 128)
    y4 = _conv_nhwc(x4, c4_W, c4_b, bn4_g, bn4_b, 4, 64)
    x5 = y4.reshape(_B, 2, 32, 2, 128).reshape(_B, 2, 32, 256)
    y5 = _conv_nhwc(x5, c5_W, c5_b, bn5_g, bn5_b, 2, 32)
    xp = y5.reshape(_B, _T, 256)

    probs = _classifier(xp, sc1_W, sc1_b, sbn1_g, sbn1_b, sc2_W, sc2_b,
                        sbn2_g, sbn2_b, sfc1_W, sfc1_b, sfc2_W, sfc2_b)

    xs2 = xp.transpose(1, 0, 2).reshape(_T * _B, 256)
    out2d = _heads(xs2, probs, h_Wih1, h_Whh1, h_bih1, h_bhh1, h_Wih2,
                   h_Whh2, h_bih2, h_bhh2, h_fcW, h_fcb)
    output = out2d.reshape(_T, _B, _A1).transpose(1, 0, 2)
    return output, probs


# speedup vs baseline: 1.4732x; 1.0525x over previous
import functools

import jax
import jax.numpy as jnp
from jax.experimental import pallas as pl
from jax.experimental.pallas import tpu as pltpu

_LEAK = 0.1
_EPS = 1e-5
_B = 32
_T = 32
_HID = 256
_A1 = 129
_NS = 4


def _dg2(a, b):
    return jax.lax.dot_general(a, b, (((a.ndim - 1,), (1,)), ((), ())),
                               preferred_element_type=jnp.float32)


def _leaky(x):
    return jnp.where(x >= 0, x, _LEAK * x)


def _bn(y, g, bb, axes):
    m = jnp.mean(y, axis=axes, keepdims=True)
    v = jnp.mean((y - m) * (y - m), axis=axes, keepdims=True)
    return g * (y - m) / jnp.sqrt(v + _EPS) + bb


def _conv1_kernel(x_ref, w_ref, b_ref, g_ref, bb_ref, o_ref):
    acc = jnp.zeros((_B, 32, 256), jnp.float32)
    for dh in range(3):
        for dw in range(3):
            k = 3 * dh + dw
            acc = acc + x_ref[:, dh:dh + 32, dw:dw + 256] * w_ref[0][0:1, k:k + 1]
    y = _leaky(acc + b_ref[0, 0, 0])
    m = jnp.mean(y)
    v = jnp.mean((y - m) * (y - m))
    y = g_ref[0, 0, 0] * (y - m) / jnp.sqrt(v + _EPS) + bb_ref[0, 0, 0]
    y = jnp.max(y.reshape(_B, 16, 2, 256), axis=2)
    o_ref[...] = y[None]


def _conv1(x3, c1_W, c1_b, bn1_g, bn1_b):
    return pl.pallas_call(
        _conv1_kernel,
        grid=(16,),
        in_specs=[
            pl.BlockSpec((_B, 34, 258), lambda c: (0, 0, 0)),
            pl.BlockSpec((1, 1, 9), lambda c: (c, 0, 0)),
            pl.BlockSpec((1, 1, 1), lambda c: (c, 0, 0)),
            pl.BlockSpec((1, 1, 1), lambda c: (c, 0, 0)),
            pl.BlockSpec((1, 1, 1), lambda c: (c, 0, 0)),
        ],
        out_specs=pl.BlockSpec((1, _B, 16, 256), lambda c: (c, 0, 0, 0)),
        out_shape=jax.ShapeDtypeStruct((16, _B, 16, 256), jnp.float32),
    )(x3, c1_W.reshape(16, 1, 9), c1_b.reshape(16, 1, 1),
      bn1_g.reshape(16, 1, 1), bn1_b.reshape(16, 1, 1))


def _conv_bhw_kernel(x_ref, w_ref, b_ref, g_ref, bb_ref, o_ref, *, H, W, Cin,
                     Cout, parity_in):
    if parity_in:
        xm = jnp.maximum(x_ref[:, :, 0], x_ref[:, :, 1])
    else:
        xm = x_ref[...]
    acc = jnp.zeros((_B * H, Cout, W), jnp.float32)
    for dh in range(3):
        for dw in range(3):
            xs = xm[:, dh:dh + H, :, dw:dw + W].reshape(_B * H, Cin, W)
            wb = jnp.broadcast_to(w_ref[dh, dw][None], (_B * H, Cout, Cin))
            acc = acc + jax.lax.dot_general(
                wb, xs, (((2,), (1,)), ((0,), (0,))),
                preferred_element_type=jnp.float32)
    y = _bn(_leaky(acc + b_ref[...]), g_ref[...], bb_ref[...], (0, 2))
    y = y.reshape(_B, H, Cout, W).reshape(_B, H // 2, 2, Cout, W)
    o_ref[...] = jnp.max(y, axis=2)


def _conv_bhw(x_pad, W_oihw, b, g, bb, H, W, parity_in):
    Cout, Cin = W_oihw.shape[0], W_oihw.shape[1]
    w = W_oihw.transpose(2, 3, 0, 1)
    return pl.pallas_call(
        functools.partial(_conv_bhw_kernel, H=H, W=W, Cin=Cin, Cout=Cout,
                          parity_in=parity_in),
        out_shape=jax.ShapeDtypeStruct((_B, H // 2, Cout, W), jnp.float32),
    )(x_pad, w, b.reshape(1, -1, 1), g.reshape(1, -1, 1), bb.reshape(1, -1, 1))


def _conv_nhwc_kernel(x_ref, w_ref, b_ref, g_ref, bb_ref, o_ref, pad_sc, *,
                      H, W, Cin, Cout):
    xin = jnp.maximum(x_ref[..., :Cin], x_ref[..., Cin:])
    pad_sc[...] = jnp.zeros((_B, H + 2, W + 2, Cin), jnp.float32)
    pad_sc[:, 1:H + 1, 1:W + 1, :] = xin
    x = pad_sc[...]
    acc = jnp.zeros((_B, H, W, Cout), jnp.float32)
    for dh in range(3):
        for dw in range(3):
            xs = x[:, dh:dh + H, dw:dw + W, :]
            acc = acc + jax.lax.dot_general(
                xs, w_ref[dh, dw], (((3,), (0,)), ((), ())),
                preferred_element_type=jnp.float32)
    y = _bn(_leaky(acc + b_ref[...]), g_ref[...], bb_ref[...], (0, 1, 2))
    y = jnp.max(y.reshape(_B, H // 2, 2, W, Cout), axis=2)
    o_ref[...] = y


def _conv_nhwc(x_packed, W_oihw, b, g, bb, H, W):
    Cout, Cin = W_oihw.shape[0], W_oihw.shape[1]
    w = W_oihw.transpose(2, 3, 1, 0)
    return pl.pallas_call(
        functools.partial(_conv_nhwc_kernel, H=H, W=W, Cin=Cin, Cout=Cout),
        out_shape=jax.ShapeDtypeStruct((_B, H // 2, W, Cout), jnp.float32),
        scratch_shapes=[pltpu.VMEM((_B, H + 2, W + 2, Cin), jnp.float32)],
    )(x_packed, w, b.reshape(1, 1, 1, -1), g.reshape(1, 1, 1, -1),
      bb.reshape(1, 1, 1, -1))


def _classifier_kernel(x_ref, w1_ref, b1_ref, g1_ref, bb1_ref, w2_ref, b2_ref,
                       g2_ref, bb2_ref, f1w_ref, f1b_ref, f2w_ref, f2b_ref,
                       o_ref):
    x = x_ref[...]
    acc = jnp.zeros((_B, 29, 128), jnp.float32)
    for j in range(4):
        acc = acc + jax.lax.dot_general(
            x[:, j:j + 29, :], w1_ref[j], (((2,), (0,)), ((), ())),
            preferred_element_type=jnp.float32)
    y = _bn(_leaky(acc + b1_ref[...]), g1_ref[...], bb1_ref[...], (0, 1))
    y = jnp.concatenate(
        [jnp.max(y[:, 4 * k:4 * k + 4, :], axis=1, keepdims=True)
         for k in range(7)], axis=1)
    acc2 = jnp.zeros((_B, 4, 64), jnp.float32)
    for j in range(4):
        acc2 = acc2 + jax.lax.dot_general(
            y[:, j:j + 4, :], w2_ref[j], (((2,), (0,)), ((), ())),
            preferred_element_type=jnp.float32)
    z = _bn(_leaky(acc2 + b2_ref[...]), g2_ref[...], bb2_ref[...], (0, 1))
    z = jnp.max(z, axis=1)
    h = jnp.maximum(_dg2(z, f1w_ref[...]) + f1b_ref[...], 0.0)
    logits = _dg2(h, f2w_ref[...]) + f2b_ref[...]
    mx = jnp.max(logits, axis=1, keepdims=True)
    e = jnp.exp(logits - mx)
    o_ref[...] = e / jnp.sum(e, axis=1, keepdims=True)


def _classifier(xp, sc1_W, sc1_b, sbn1_g, sbn1_b, sc2_W, sc2_b, sbn2_g,
                sbn2_b, sfc1_W, sfc1_b, sfc2_W, sfc2_b):
    return pl.pallas_call(
        _classifier_kernel,
        out_shape=jax.ShapeDtypeStruct((_B, _NS), jnp.float32),
    )(xp, sc1_W.transpose(2, 1, 0), sc1_b.reshape(1, 1, -1),
      sbn1_g.reshape(1, 1, -1), sbn1_b.reshape(1, 1, -1),
      sc2_W.transpose(2, 1, 0), sc2_b.reshape(1, 1, -1),
      sbn2_g.reshape(1, 1, -1), sbn2_b.reshape(1, 1, -1),
      sfc1_W, sfc1_b.reshape(1, -1), sfc2_W, sfc2_b.reshape(1, -1))


def _dgk(a, w):
    return jax.lax.dot_general(a, w, (((1,), (0,)), ((), ())),
                               preferred_element_type=jnp.float32)


def _gates(g, c):
    i = jax.nn.sigmoid(g[:, 0:256])
    f = jax.nn.sigmoid(g[:, 256:512])
    gg = jnp.tanh(g[:, 512:768])
    o = jax.nn.sigmoid(g[:, 768:1024])
    c = f * c + i * gg
    return o * jnp.tanh(c), c


def _heads_kernel(xs_ref, p_ref, w1_ref, wh1_ref, b1_ref, w2_ref, wh2_ref,
                  b2_ref, fcw_ref, fcb_ref, o_ref, ih_sc, y_sc):
    N = _T * _B
    X = xs_ref[...]

    p = p_ref[...]
    pm = jnp.max(p, axis=1, keepdims=True)
    iota = jax.lax.broadcasted_iota(jnp.int32, (_B, _NS), 1)
    sidx = jnp.min(jnp.where(p >= pm, iota, _NS), axis=1, keepdims=True)
    ri = jax.lax.broadcasted_iota(jnp.int32, (N, _B), 0)
    bj = jax.lax.broadcasted_iota(jnp.int32, (N, _B), 1)
    onehot = ((ri % _B) == bj).astype(jnp.float32)
    sidx_rows = jax.lax.dot_general(
        onehot, sidx.astype(jnp.float32), (((1,), (0,)), ((), ())),
        preferred_element_type=jnp.float32)
    mrows = [(sidx_rows == float(s)).astype(jnp.float32) for s in range(4)]
    mb = [(sidx == s).astype(jnp.float32) for s in range(4)]

    def expand_rows(v):
        return jnp.concatenate([m * v for m in mrows], axis=1)

    def expand_b(v):
        return jnp.concatenate([m * v for m in mb], axis=1)

    def bias_sel(b_ref):
        out = mrows[0] * b_ref[0:1]
        for s in range(1, 4):
            out = out + mrows[s] * b_ref[s:s + 1]
        return out

    ih_sc[...] = _dgk(expand_rows(X), w1_ref[...]) + bias_sel(b1_ref)

    wh1 = wh1_ref[...]

    def step1(t, hc):
        h, c = hc
        g = ih_sc[pl.ds(t * _B, _B), :] + _dgk(expand_b(h), wh1)
        h, c = _gates(g, c)
        y_sc[pl.ds(t * _B, _B), :] = h
        return (h, c)

    z = jnp.zeros((_B, _HID), jnp.float32)
    h1, c1 = jax.lax.fori_loop(0, _T, step1, (z, z))

    ih_sc[...] = _dgk(expand_rows(y_sc[...]), w2_ref[...]) + bias_sel(b2_ref)

    wh2 = wh2_ref[...]

    def step2(t, hc):
        h, c = hc
        g = ih_sc[pl.ds(t * _B, _B), :] + _dgk(expand_b(h), wh2)
        h, c = _gates(g, c)
        y_sc[pl.ds(t * _B, _B), :] = h
        return (h, c)

    jax.lax.fori_loop(0, _T, step2, (h1, c1))

    logits = _dgk(expand_rows(y_sc[...]), fcw_ref[...]) + bias_sel(fcb_ref)
    mx = jnp.max(logits, axis=1, keepdims=True)
    e = jnp.exp(logits - mx)
    o_ref[...] = e / jnp.sum(e, axis=1, keepdims=True)


def _heads(xs2, probs, h_Wih1, h_Whh1, h_bih1, h_bhh1, h_Wih2, h_Whh2,
           h_bih2, h_bhh2, h_fcW, h_fcb):
    N = _T * _B
    stack = lambda w: w.transpose(0, 2, 1).reshape(_NS * _HID, -1)
    return pl.pallas_call(
        _heads_kernel,
        out_shape=jax.ShapeDtypeStruct((N, _A1), jnp.float32),
        scratch_shapes=[
            pltpu.VMEM((N, 4 * _HID), jnp.float32),
            pltpu.VMEM((N, _HID), jnp.float32),
        ],
    )(xs2, probs, stack(h_Wih1), stack(h_Whh1), h_bih1 + h_bhh1,
      stack(h_Wih2), stack(h_Whh2), h_bih2 + h_bhh2, stack(h_fcW), h_fcb)


def kernel(x, c1_W, c1_b, bn1_g, bn1_b, c2_W, c2_b, bn2_g, bn2_b, c3_W, c3_b,
           bn3_g, bn3_b, c4_W, c4_b, bn4_g, bn4_b, c5_W, c5_b, bn5_g, bn5_b,
           sc1_W, sc1_b, sbn1_g, sbn1_b, sc2_W, sc2_b, sbn2_g, sbn2_b, sfc1_W,
           sfc1_b, sfc2_W, sfc2_b, h_Wih1, h_Whh1, h_bih1, h_bhh1, h_Wih2,
           h_Whh2, h_bih2, h_bhh2, h_fcW, h_fcb):
    x3 = jnp.pad(x[:, 0], ((0, 0), (1, 1), (1, 1)))
    y1 = _conv1(x3, c1_W, c1_b, bn1_g, bn1_b)
    x2 = y1.transpose(1, 2, 0, 3).reshape(_B, 16, 16, 128, 2)
    x2 = x2.transpose(0, 1, 4, 2, 3)
    x2 = jnp.pad(x2, ((0, 0), (1, 1), (0, 0), (0, 0), (1, 1)))
    y2 = _conv_bhw(x2, c2_W, c2_b, bn2_g, bn2_b, 16, 128, True)
    x3b = jnp.pad(y2, ((0, 0), (1, 1), (0, 0), (1, 1)))
    y3 = _conv_bhw(x3b, c3_W, c3_b, bn3_g, bn3_b, 8, 128, False)
    x4 = y3.transpose(0, 1, 3, 2).reshape(_B, 4, 64, 2, 64)
    x4 = x4.reshape(_B, 4, 64, 128)
    y4 = _conv_nhwc(x4, c4_W, c4_b, bn4_g, bn4_b, 4, 64)
    x5 = y4.reshape(_B, 2, 32, 2, 128).reshape(_B, 2, 32, 256)
    y5 = _conv_nhwc(x5, c5_W, c5_b, bn5_g, bn5_b, 2, 32)
    xp = y5.reshape(_B, _T, 256)

    probs = _classifier(xp, sc1_W, sc1_b, sbn1_g, sbn1_b, sc2_W, sc2_b,
                        sbn2_g, sbn2_b, sfc1_W, sfc1_b, sfc2_W, sfc2_b)

    xs2 = xp.transpose(1, 0, 2).reshape(_T * _B, 256)
    out2d = _heads(xs2, probs, h_Wih1, h_Whh1, h_bih1, h_bhh1, h_Wih2,
                   h_Whh2, h_bih2, h_bhh2, h_fcW, h_fcb)
    output = out2d.reshape(_T, _B, _A1).transpose(1, 0, 2)
    return output, probs
